# baseline (speedup 1.0000x reference)
import functools

import jax
import jax.numpy as jnp
from jax import lax
from jax.experimental import pallas as pl
from jax.experimental.pallas import tpu as pltpu

F32 = jnp.float32
BF16 = jnp.bfloat16
_HI = lax.Precision.HIGHEST


def _dot(a, b, trans_b=False, exact=False):
    dn = (((1,), (1 if trans_b else 0,)), ((), ()))
    if exact:
        return lax.dot_general(a, b, dn, precision=_HI,
                               preferred_element_type=F32)
    return lax.dot_general(a.astype(BF16), b.astype(BF16), dn,
                           preferred_element_type=F32)


def _mm_body(a_ref, w_ref, b_ref, o_ref, *, act):
    a = a_ref[0] if a_ref.ndim == 3 else a_ref[...]
    w = w_ref[0] if w_ref.ndim == 3 else w_ref[...]
    y = _dot(a, w) + b_ref[...]
    if act == "relu":
        y = jnp.maximum(y, 0.0)
    elif act == "tanh":
        y = jnp.tanh(y)
    if o_ref.ndim == 3:
        o_ref[0] = y
    else:
        o_ref[...] = y


def _mm(a, w, bias, act="none", mb=1792):
    M, K = a.shape
    N = w.shape[1]
    nblk = M // mb
    assert nblk * mb == M
    return pl.pallas_call(
        functools.partial(_mm_body, act=act),
        grid=(nblk,),
        in_specs=[
            pl.BlockSpec((mb, K), lambda i: (i, 0)),
            pl.BlockSpec((K, N), lambda i: (0, 0)),
            pl.BlockSpec((1, N), lambda i: (0, 0)),
        ],
        out_specs=pl.BlockSpec((mb, N), lambda i: (i, 0)),
        out_shape=jax.ShapeDtypeStruct((M, N), F32),
    )(a, w, bias.reshape(1, N))


def _mm_batched(a, w, bias, act="none", mb=1792):
    P, M, K = a.shape
    N = w.shape[2]
    nblk = M // mb
    assert nblk * mb == M
    return pl.pallas_call(
        functools.partial(_mm_body, act=act),
        grid=(P, nblk),
        in_specs=[
            pl.BlockSpec((1, mb, K), lambda p, i: (p, i, 0)),
            pl.BlockSpec((1, K, N), lambda p, i: (p, 0, 0)),
            pl.BlockSpec((1, N), lambda p, i: (0, 0)),
        ],
        out_specs=pl.BlockSpec((1, mb, N), lambda p, i: (p, i, 0)),
        out_shape=jax.ShapeDtypeStruct((P, M, N), F32),
    )(a, w, bias.reshape(1, N))


def _vq_body(tok_ref, cb_ref, q_ref, idx_ref, loss_ref, *, nq, v, tb):
    i = pl.program_id(0)

    @pl.when(i == 0)
    def _init():
        loss_ref[...] = jnp.zeros_like(loss_ref)

    r = tok_ref[...]
    quant = jnp.zeros_like(r)
    iota = lax.broadcasted_iota(jnp.int32, (tb, v), 1)
    for q in range(nq):
        cb = cb_ref[q]
        rn = jnp.sum(r * r, axis=1, keepdims=True)
        cn = jnp.sum(cb * cb, axis=1)[None, :]
        d = rn - 2.0 * _dot(r, cb, trans_b=True) + cn
        m = jnp.min(d, axis=1, keepdims=True)
        idx = jnp.min(jnp.where(d == m, iota, v), axis=1)
        onehot = (iota == idx[:, None]).astype(F32)
        qv = _dot(onehot, cb, exact=True)
        r = r - qv
        quant = quant + qv
        idx_ref[pl.ds(q, 1), :] = idx[None, :]
        part = jnp.sum(r * r, axis=0)
        loss_ref[pl.ds(q, 1), :] += part.reshape(-1, 128).sum(axis=0)[None, :]
    q_ref[...] = quant


def _vq(tokens, codebooks, tb=896):
    T, C = tokens.shape
    NQ, V, _ = codebooks.shape
    nblk = T // tb
    assert nblk * tb == T
    return pl.pallas_call(
        functools.partial(_vq_body, nq=NQ, v=V, tb=tb),
        grid=(nblk,),
        in_specs=[
            pl.BlockSpec((tb, C), lambda i: (i, 0)),
            pl.BlockSpec((NQ, V, C), lambda i: (0, 0, 0)),
        ],
        out_specs=[
            pl.BlockSpec((tb, C), lambda i: (i, 0)),
            pl.BlockSpec((NQ, tb), lambda i: (0, i)),
            pl.BlockSpec((NQ, 128), lambda i: (0, 0)),
        ],
        out_shape=[
            jax.ShapeDtypeStruct((T, C), F32),
            jax.ShapeDtypeStruct((NQ, T), jnp.int32),
            jax.ShapeDtypeStruct((NQ, 128), F32),
        ],
    )(tokens, codebooks)


def _patches_s2k4(x_nhwc, out_hw):
    s = 2 * out_hw - 1
    cols = [
        x_nhwc[:, dy : dy + s : 2, dx : dx + s : 2, :]
        for dy in range(4)
        for dx in range(4)
    ]
    p = jnp.concatenate(cols, axis=-1)
    B = x_nhwc.shape[0]
    return p.reshape(B * out_hw * out_hw, -1)


def _patches_s1k3(x_nhwc, out_hw):
    cols = [
        x_nhwc[:, dy : dy + out_hw, dx : dx + out_hw, :]
        for dy in range(3)
        for dx in range(3)
    ]
    p = jnp.concatenate(cols, axis=-1)
    B = x_nhwc.shape[0]
    return p.reshape(B * out_hw * out_hw, -1)


def _pad1(x_nhwc):
    return jnp.pad(x_nhwc, ((0, 0), (1, 1), (1, 1), (0, 0)))


_TAPS = ((3, 1), (2, 0))


def _convt_phase_patches(x_nhwc, out_hw_in):
    B = x_nhwc.shape[0]
    phases = []
    for py in range(2):
        for px in range(2):
            cols = [
                x_nhwc[:, py + wy : py + wy + out_hw_in,
                       px + wx : px + wx + out_hw_in, :]
                for wy in range(2)
                for wx in range(2)
            ]
            phases.append(
                jnp.concatenate(cols, axis=-1).reshape(B * out_hw_in * out_hw_in, -1)
            )
    return jnp.stack(phases, axis=0)


def _convt_phase_weights(w, n_pad):
    cin, cout = w.shape[0], w.shape[1]
    ws = []
    for py in range(2):
        for px in range(2):
            sub = w[:, :, list(_TAPS[py]), :][:, :, :, list(_TAPS[px])]
            ws.append(sub.transpose(2, 3, 0, 1).reshape(4 * cin, cout))
    wp = jnp.stack(ws, axis=0)
    if n_pad > cout:
        wp = jnp.pad(wp, ((0, 0), (0, 0), (0, n_pad - cout)))
    return wp


def _interleave_phases(y, B, hin, cout):
    y = y[..., :cout].reshape(2, 2, B, hin, hin, cout)
    y = y.transpose(2, 3, 0, 4, 1, 5)
    return y.reshape(B, 2 * hin, 2 * hin, cout)


def kernel(x, w1, b1, w2, b2, w3, b3, w4, b4, codebooks, dw0, db0,
           dtw1, dtb1, dtw2, dtb2):
    B = x.shape[0]
    xh = x.transpose(0, 2, 3, 1)

    p1 = _patches_s2k4(_pad1(xh), 112)
    wm1 = w1.transpose(2, 3, 1, 0).reshape(48, 64)
    y1 = _mm(p1, wm1, b1, act="relu", mb=1792)
    y1 = y1.reshape(B, 112, 112, 64)

    p2 = _patches_s2k4(_pad1(y1), 56)
    wm2 = w2.transpose(2, 3, 1, 0).reshape(1024, 128)
    y2 = _mm(p2, wm2, b2, act="relu", mb=896)
    y2 = y2.reshape(B, 56, 56, 128)

    p3 = _patches_s1k3(_pad1(y2), 56)
    wm3 = w3.transpose(2, 3, 1, 0).reshape(1152, 256)
    y3 = _mm(p3, wm3, b3, act="relu", mb=896)
    y3 = y3.reshape(B, 56, 56, 256)

    p4 = _patches_s1k3(_pad1(y3), 56)
    wm4 = w4.transpose(2, 3, 1, 0).reshape(2304, 256)
    tokens = _mm(p4, wm4, b4, act="none", mb=896)

    quant, idx, loss_part = _vq(tokens, codebooks, tb=896)
    T, C = tokens.shape
    commit_loss = loss_part.sum(axis=1) / (T * C)
    indices = idx.reshape(4, B, 56, 56).transpose(1, 0, 2, 3)

    qmap_nhwc = quant.reshape(B, 56, 56, C)
    qmap = qmap_nhwc.transpose(0, 3, 1, 2)

    pd = _patches_s1k3(_pad1(qmap_nhwc), 56)
    wmd = dw0.transpose(2, 3, 1, 0).reshape(2304, 128)
    r0 = _mm(pd, wmd, db0, act="relu", mb=896)
    r0 = r0.reshape(B, 56, 56, 128)

    pt1 = _convt_phase_patches(_pad1(r0), 56)
    wt1 = _convt_phase_weights(dtw1, 128)
    bt1 = jnp.pad(dtb1, (0, 64))
    r1 = _mm_batched(pt1, wt1, bt1, act="relu", mb=896)
    r1 = _interleave_phases(r1, B, 56, 64)

    pt2 = _convt_phase_patches(_pad1(r1), 112)
    wt2 = _convt_phase_weights(dtw2, 128)
    bt2 = jnp.pad(dtb2, (0, 125))
    r2 = _mm_batched(pt2, wt2, bt2, act="tanh", mb=1792)
    recon = _interleave_phases(r2, B, 112, 3)
    recon = recon.transpose(0, 3, 1, 2)

    return recon, indices, commit_loss, qmap


# speedup vs baseline: 1.4463x; 1.4463x over previous
import functools

import jax
import jax.numpy as jnp
from jax import lax
from jax.experimental import pallas as pl

F32 = jnp.float32
BF16 = jnp.bfloat16


def _dot(a, b, trans_b=False, exact=False):
    dn = (((1,), (1 if trans_b else 0,)), ((), ()))
    if exact:
        return lax.dot_general(a, b, dn, precision=lax.Precision.HIGHEST,
                               preferred_element_type=F32)
    return lax.dot_general(a.astype(BF16), b.astype(BF16), dn,
                           preferred_element_type=F32)


def _mm_body(a_ref, w_ref, b_ref, o_ref, *, act):
    y = _dot(a_ref[...], w_ref[...]) + b_ref[...]
    if act == "relu":
        y = jnp.maximum(y, 0.0)
    elif act == "tanh":
        y = jnp.tanh(y)
    o_ref[...] = y


def _mm(a, w, bias, act="none", mb=1792):
    M, K = a.shape
    N = w.shape[1]
    nblk = M // mb
    assert nblk * mb == M
    return pl.pallas_call(
        functools.partial(_mm_body, act=act),
        grid=(nblk,),
        in_specs=[
            pl.BlockSpec((mb, K), lambda i: (i, 0)),
            pl.BlockSpec((K, N), lambda i: (0, 0)),
            pl.BlockSpec((1, N), lambda i: (0, 0)),
        ],
        out_specs=pl.BlockSpec((mb, N), lambda i: (i, 0)),
        out_shape=jax.ShapeDtypeStruct((M, N), F32),
    )(a, w, bias.reshape(1, N))


def _vq_body(tok_ref, cb_ref, q_ref, idx_ref, loss_ref, *, nq, v, tb):
    i = pl.program_id(0)

    @pl.when(i == 0)
    def _init():
        loss_ref[...] = jnp.zeros_like(loss_ref)

    r = tok_ref[...]
    quant = jnp.zeros_like(r)
    iota = lax.broadcasted_iota(jnp.int32, (tb, v), 1)
    for q in range(nq):
        cb = cb_ref[q]
        rn = jnp.sum(r * r, axis=1, keepdims=True)
        cn = jnp.sum(cb * cb, axis=1)[None, :]
        d = rn - 2.0 * _dot(r, cb, trans_b=True) + cn
        m = jnp.min(d, axis=1, keepdims=True)
        idx = jnp.min(jnp.where(d == m, iota, v), axis=1)
        onehot = (iota == idx[:, None]).astype(F32)
        qv = _dot(onehot, cb, exact=True)
        r = r - qv
        quant = quant + qv
        idx_ref[pl.ds(q, 1), :] = idx[None, :]
        part = jnp.sum(r * r, axis=0)
        loss_ref[pl.ds(q, 1), :] += part.reshape(-1, 128).sum(axis=0)[None, :]
    q_ref[...] = quant


def _vq(tokens, codebooks, tb=896):
    T, C = tokens.shape
    NQ, V, _ = codebooks.shape
    nblk = T // tb
    assert nblk * tb == T
    return pl.pallas_call(
        functools.partial(_vq_body, nq=NQ, v=V, tb=tb),
        grid=(nblk,),
        in_specs=[
            pl.BlockSpec((tb, C), lambda i: (i, 0)),
            pl.BlockSpec((NQ, V, C), lambda i: (0, 0, 0)),
        ],
        out_specs=[
            pl.BlockSpec((tb, C), lambda i: (i, 0)),
            pl.BlockSpec((NQ, tb), lambda i: (0, i)),
            pl.BlockSpec((NQ, 128), lambda i: (0, 0)),
        ],
        out_shape=[
            jax.ShapeDtypeStruct((T, C), F32),
            jax.ShapeDtypeStruct((NQ, T), jnp.int32),
            jax.ShapeDtypeStruct((NQ, 128), F32),
        ],
    )(tokens, codebooks)


def _patches_s2k4(x_nhwc, out_hw):
    B, H, W, C = x_nhwc.shape
    xp = jnp.pad(x_nhwc, ((0, 0), (1, 1), (1, 1), (0, 0))).astype(BF16)
    s = 2 * out_hw - 1
    cols = [xp[:, dy : dy + s : 2, dx : dx + s : 2, :]
            for dy in range(4) for dx in range(4)]
    return jnp.concatenate(cols, axis=-1).reshape(B * out_hw * out_hw, 16 * C)


def _w_s2k4(w):
    O, C = w.shape[0], w.shape[1]
    return w.transpose(2, 3, 1, 0).reshape(16 * C, O).astype(BF16)


def _patches_s1k3(x_nhwc, out_hw):
    B, H, W, C = x_nhwc.shape
    xp = jnp.pad(x_nhwc, ((0, 0), (1, 1), (1, 1), (0, 0))).astype(BF16)
    cols = [xp[:, dy : dy + out_hw, dx : dx + out_hw, :]
            for dy in range(3) for dx in range(3)]
    return jnp.concatenate(cols, axis=-1).reshape(B * out_hw * out_hw, 9 * C)


def _w_s1k3(w):
    O, C = w.shape[0], w.shape[1]
    return w.transpose(2, 3, 1, 0).reshape(9 * C, O).astype(BF16)


_TAPS = ((3, 1), (2, 0))


def _w_convt(w, n_pad):
    cin, cout = w.shape[0], w.shape[1]
    blocks = []
    for dy in range(3):
        for dx in range(3):
            phase_cols = []
            for py in range(2):
                for px in range(2):
                    wy, wx = dy - py, dx - px
                    if 0 <= wy <= 1 and 0 <= wx <= 1:
                        sub = w[:, :, _TAPS[py][wy], _TAPS[px][wx]]
                    else:
                        sub = jnp.zeros((cin, cout), F32)
                    phase_cols.append(sub)
            blocks.append(jnp.concatenate(phase_cols, axis=1))
    wm = jnp.concatenate(blocks, axis=0)
    if n_pad > 4 * cout:
        wm = jnp.pad(wm, ((0, 0), (0, n_pad - 4 * cout)))
    return wm.astype(BF16)


def _interleave_phases(y, B, hin, cout):
    y = y[:, : 4 * cout].reshape(B, hin, hin, 2, 2, cout)
    y = y.transpose(0, 1, 3, 2, 4, 5)
    return y.reshape(B, 2 * hin, 2 * hin, cout)


def kernel(x, w1, b1, w2, b2, w3, b3, w4, b4, codebooks, dw0, db0,
           dtw1, dtb1, dtw2, dtb2):
    B = x.shape[0]
    xh = x.transpose(0, 2, 3, 1)

    y1 = _mm(_patches_s2k4(xh, 112), _w_s2k4(w1), b1, act="relu", mb=1792)
    y1 = y1.reshape(B, 112, 112, 64)

    y2 = _mm(_patches_s2k4(y1, 56), _w_s2k4(w2), b2, act="relu", mb=896)
    y2 = y2.reshape(B, 56, 56, 128)

    y3 = _mm(_patches_s1k3(y2, 56), _w_s1k3(w3), b3, act="relu", mb=896)
    y3 = y3.reshape(B, 56, 56, 256)

    tokens = _mm(_patches_s1k3(y3, 56), _w_s1k3(w4), b4, act="none", mb=896)

    quant, idx, loss_part = _vq(tokens, codebooks, tb=896)
    T, C = tokens.shape
    commit_loss = loss_part.sum(axis=1) / (T * C)
    indices = idx.reshape(4, B, 56, 56).transpose(1, 0, 2, 3)

    qmap_nhwc = quant.reshape(B, 56, 56, C)
    qmap = qmap_nhwc.transpose(0, 3, 1, 2)

    r0 = _mm(_patches_s1k3(qmap_nhwc, 56), _w_s1k3(dw0), db0, act="relu",
             mb=896)
    r0 = r0.reshape(B, 56, 56, 128)

    bt1 = jnp.pad(jnp.tile(dtb1, 4), (0, 0))
    r1 = _mm(_patches_s1k3(r0, 56), _w_convt(dtw1, 256), bt1, act="relu",
             mb=896)
    r1 = _interleave_phases(r1, B, 56, 64)

    bt2 = jnp.pad(jnp.tile(dtb2, 4), (0, 116))
    r2 = _mm(_patches_s1k3(r1, 112), _w_convt(dtw2, 128), bt2, act="tanh",
             mb=1792)
    recon = _interleave_phases(r2, B, 112, 3)
    recon = recon.transpose(0, 3, 1, 2)

    return recon, indices, commit_loss, qmap


# speedup vs baseline: 1.9597x; 1.3550x over previous
import functools

import jax
import jax.numpy as jnp
from jax import lax
from jax.experimental import pallas as pl

F32 = jnp.float32
BF16 = jnp.bfloat16


def _dot(a, b, trans_b=False, exact=False):
    dn = (((1,), (1 if trans_b else 0,)), ((), ()))
    if exact:
        return lax.dot_general(a, b, dn, precision=lax.Precision.HIGHEST,
                               preferred_element_type=F32)
    return lax.dot_general(a.astype(BF16), b.astype(BF16), dn,
                           preferred_element_type=F32)


def _mm_body(a_ref, w_ref, b_ref, o_ref, *, act):
    y = _dot(a_ref[...], w_ref[...]) + b_ref[...]
    if act == "relu":
        y = jnp.maximum(y, 0.0)
    elif act == "tanh":
        y = jnp.tanh(y)
    o_ref[...] = y


def _mm(a, w, bias, act="none", mb=1792):
    M, K = a.shape
    N = w.shape[1]
    nblk = M // mb
    assert nblk * mb == M
    return pl.pallas_call(
        functools.partial(_mm_body, act=act),
        grid=(nblk,),
        in_specs=[
            pl.BlockSpec((mb, K), lambda i: (i, 0)),
            pl.BlockSpec((K, N), lambda i: (0, 0)),
            pl.BlockSpec((1, N), lambda i: (0, 0)),
        ],
        out_specs=pl.BlockSpec((mb, N), lambda i: (i, 0)),
        out_shape=jax.ShapeDtypeStruct((M, N), F32),
    )(a, w, bias.reshape(1, N))


def _vq_body(tok_ref, cb_ref, q_ref, idx_ref, loss_ref, *, nq, v, tb):
    i = pl.program_id(0)

    @pl.when(i == 0)
    def _init():
        loss_ref[...] = jnp.zeros_like(loss_ref)

    r = tok_ref[...]
    quant = jnp.zeros_like(r)
    iota = lax.broadcasted_iota(jnp.int32, (tb, v), 1)
    for q in range(nq):
        cb = cb_ref[q]
        rn = jnp.sum(r * r, axis=1, keepdims=True)
        cn = jnp.sum(cb * cb, axis=1)[None, :]
        d = rn - 2.0 * _dot(r, cb, trans_b=True) + cn
        m = jnp.min(d, axis=1, keepdims=True)
        idx = jnp.min(jnp.where(d == m, iota, v), axis=1)
        onehot = (iota == idx[:, None]).astype(F32)
        qv = _dot(onehot, cb, exact=True)
        r = r - qv
        quant = quant + qv
        idx_ref[pl.ds(q, 1), :] = idx[None, :]
        part = jnp.sum(r * r, axis=0)
        loss_ref[pl.ds(q, 1), :] += part.reshape(-1, 128).sum(axis=0)[None, :]
    q_ref[...] = quant


def _vq(tokens, codebooks, tb=896):
    T, C = tokens.shape
    NQ, V, _ = codebooks.shape
    nblk = T // tb
    assert nblk * tb == T
    return pl.pallas_call(
        functools.partial(_vq_body, nq=NQ, v=V, tb=tb),
        grid=(nblk,),
        in_specs=[
            pl.BlockSpec((tb, C), lambda i: (i, 0)),
            pl.BlockSpec((NQ, V, C), lambda i: (0, 0, 0)),
        ],
        out_specs=[
            pl.BlockSpec((tb, C), lambda i: (i, 0)),
            pl.BlockSpec((NQ, tb), lambda i: (0, i)),
            pl.BlockSpec((NQ, 128), lambda i: (0, 0)),
        ],
        out_shape=[
            jax.ShapeDtypeStruct((T, C), F32),
            jax.ShapeDtypeStruct((NQ, T), jnp.int32),
            jax.ShapeDtypeStruct((NQ, 128), F32),
        ],
    )(tokens, codebooks)


def _patches_s2k4(x_nhwc, out_hw):
    B, H, W, C = x_nhwc.shape
    xp = jnp.pad(x_nhwc, ((0, 0), (1, 1), (1, 1), (0, 0))).astype(BF16)
    hp = out_hw + 1
    pl4 = xp.reshape(B, hp, 2, hp, 2, C).transpose(0, 2, 4, 1, 3, 5)
    cols = [
        pl4[:, dy % 2, dx % 2,
            dy // 2 : dy // 2 + out_hw, dx // 2 : dx // 2 + out_hw, :]
        for dy in range(4) for dx in range(4)
    ]
    return jnp.concatenate(cols, axis=-1).reshape(B * out_hw * out_hw, 16 * C)


def _w_s2k4(w):
    O, C = w.shape[0], w.shape[1]
    return w.transpose(2, 3, 1, 0).reshape(16 * C, O).astype(BF16)


def _patches_s1k3(x_nhwc, out_hw):
    B, H, W, C = x_nhwc.shape
    xp = jnp.pad(x_nhwc, ((0, 0), (1, 1), (1, 1), (0, 0))).astype(BF16)
    cols = [xp[:, dy : dy + out_hw, dx : dx + out_hw, :]
            for dy in range(3) for dx in range(3)]
    return jnp.concatenate(cols, axis=-1).reshape(B * out_hw * out_hw, 9 * C)


def _w_s1k3(w):
    O, C = w.shape[0], w.shape[1]
    return w.transpose(2, 3, 1, 0).reshape(9 * C, O).astype(BF16)


_TAPS = ((3, 1), (2, 0))


def _w_convt(w, n_pad):
    cin, cout = w.shape[0], w.shape[1]
    blocks = []
    for dy in range(3):
        for dx in range(3):
            phase_cols = []
            for py in range(2):
                for px in range(2):
                    wy, wx = dy - py, dx - px
                    if 0 <= wy <= 1 and 0 <= wx <= 1:
                        sub = w[:, :, _TAPS[py][wy], _TAPS[px][wx]]
                    else:
                        sub = jnp.zeros((cin, cout), F32)
                    phase_cols.append(sub)
            blocks.append(jnp.concatenate(phase_cols, axis=1))
    wm = jnp.concatenate(blocks, axis=0)
    if n_pad > 4 * cout:
        wm = jnp.pad(wm, ((0, 0), (0, n_pad - 4 * cout)))
    return wm.astype(BF16)


def _interleave_phases(y, B, hin, cout):
    y = y[:, : 4 * cout].reshape(B, hin, hin, 2, 2, cout)
    y = y.transpose(0, 1, 3, 2, 4, 5)
    return y.reshape(B, 2 * hin, 2 * hin, cout)


def kernel(x, w1, b1, w2, b2, w3, b3, w4, b4, codebooks, dw0, db0,
           dtw1, dtb1, dtw2, dtb2):
    B = x.shape[0]
    xh = x.transpose(0, 2, 3, 1)

    y1 = _mm(_patches_s2k4(xh, 112), _w_s2k4(w1), b1, act="relu", mb=1792)
    y1 = y1.reshape(B, 112, 112, 64)

    y2 = _mm(_patches_s2k4(y1, 56), _w_s2k4(w2), b2, act="relu", mb=896)
    y2 = y2.reshape(B, 56, 56, 128)

    y3 = _mm(_patches_s1k3(y2, 56), _w_s1k3(w3), b3, act="relu", mb=896)
    y3 = y3.reshape(B, 56, 56, 256)

    tokens = _mm(_patches_s1k3(y3, 56), _w_s1k3(w4), b4, act="none", mb=896)

    quant, idx, loss_part = _vq(tokens, codebooks, tb=896)
    T, C = tokens.shape
    commit_loss = loss_part.sum(axis=1) / (T * C)
    indices = idx.reshape(4, B, 56, 56).transpose(1, 0, 2, 3)

    qmap_nhwc = quant.reshape(B, 56, 56, C)
    qmap = qmap_nhwc.transpose(0, 3, 1, 2)

    r0 = _mm(_patches_s1k3(qmap_nhwc, 56), _w_s1k3(dw0), db0, act="relu",
             mb=896)
    r0 = r0.reshape(B, 56, 56, 128)

    bt1 = jnp.pad(jnp.tile(dtb1, 4), (0, 0))
    r1 = _mm(_patches_s1k3(r0, 56), _w_convt(dtw1, 256), bt1, act="relu",
             mb=896)
    r1 = _interleave_phases(r1, B, 56, 64)

    bt2 = jnp.pad(jnp.tile(dtb2, 4), (0, 116))
    r2 = _mm(_patches_s1k3(r1, 112), _w_convt(dtw2, 128), bt2, act="tanh",
             mb=1792)
    recon = _interleave_phases(r2, B, 112, 3)
    recon = recon.transpose(0, 3, 1, 2)

    return recon, indices, commit_loss, qmap


# speedup vs baseline: 2.7558x; 1.4062x over previous
import functools

import jax
import jax.numpy as jnp
from jax import lax
from jax.experimental import pallas as pl

F32 = jnp.float32
BF16 = jnp.bfloat16


def _dot(a, b, trans_b=False, exact=False):
    dn = (((1,), (1 if trans_b else 0,)), ((), ()))
    if exact:
        return lax.dot_general(a, b, dn, precision=lax.Precision.HIGHEST,
                               preferred_element_type=F32)
    return lax.dot_general(a.astype(BF16), b.astype(BF16), dn,
                           preferred_element_type=F32)


def _mm_body(a_ref, w_ref, b_ref, o_ref, *, act):
    y = _dot(a_ref[...], w_ref[...]) + b_ref[...]
    if act == "relu":
        y = jnp.maximum(y, 0.0)
    elif act == "tanh":
        y = jnp.tanh(y)
    o_ref[...] = y


def _mm(a, w, bias, act="none", mb=1792):
    M, K = a.shape
    N = w.shape[1]
    nblk = M // mb
    assert nblk * mb == M
    return pl.pallas_call(
        functools.partial(_mm_body, act=act),
        grid=(nblk,),
        in_specs=[
            pl.BlockSpec((mb, K), lambda i: (i, 0)),
            pl.BlockSpec((K, N), lambda i: (0, 0)),
            pl.BlockSpec((1, N), lambda i: (0, 0)),
        ],
        out_specs=pl.BlockSpec((mb, N), lambda i: (i, 0)),
        out_shape=jax.ShapeDtypeStruct((M, N), F32),
    )(a, w, bias.reshape(1, N))


def _vq_body(tok_ref, cb_ref, q_ref, qt_ref, idx_ref, loss_ref, *, nq, v, tb):
    i = pl.program_id(0)

    @pl.when(i == 0)
    def _init():
        loss_ref[...] = jnp.zeros_like(loss_ref)

    r = tok_ref[...]
    quant = jnp.zeros_like(r)
    iota = lax.broadcasted_iota(jnp.int32, (tb, v), 1)
    for q in range(nq):
        cb = cb_ref[q]
        rn = jnp.sum(r * r, axis=1, keepdims=True)
        cn = jnp.sum(cb * cb, axis=1)[None, :]
        d = rn - 2.0 * _dot(r, cb, trans_b=True) + cn
        idx = jnp.argmin(d, axis=1).astype(jnp.int32)
        onehot = (iota == idx[:, None]).astype(F32)
        qv = _dot(onehot, cb, exact=True)
        r = r - qv
        quant = quant + qv
        idx_ref[pl.ds(q, 1), :] = idx[None, :]
        part = jnp.sum(r * r, axis=0)
        loss_ref[pl.ds(q, 1), :] += part.reshape(-1, 128).sum(axis=0)[None, :]
    q_ref[...] = quant
    qt_ref[...] = quant.T


def _vq(tokens, codebooks, tb=896):
    T, C = tokens.shape
    NQ, V, _ = codebooks.shape
    nblk = T // tb
    assert nblk * tb == T
    return pl.pallas_call(
        functools.partial(_vq_body, nq=NQ, v=V, tb=tb),
        grid=(nblk,),
        in_specs=[
            pl.BlockSpec((tb, C), lambda i: (i, 0)),
            pl.BlockSpec((NQ, V, C), lambda i: (0, 0, 0)),
        ],
        out_specs=[
            pl.BlockSpec((tb, C), lambda i: (i, 0)),
            pl.BlockSpec((C, tb), lambda i: (0, i)),
            pl.BlockSpec((NQ, tb), lambda i: (0, i)),
            pl.BlockSpec((NQ, 128), lambda i: (0, 0)),
        ],
        out_shape=[
            jax.ShapeDtypeStruct((T, C), F32),
            jax.ShapeDtypeStruct((C, T), F32),
            jax.ShapeDtypeStruct((NQ, T), jnp.int32),
            jax.ShapeDtypeStruct((NQ, 128), F32),
        ],
    )(tokens, codebooks)


def _patches_s2k4(x_nhwc, out_hw):
    B, H, W, C = x_nhwc.shape
    xp = jnp.pad(x_nhwc, ((0, 0), (1, 1), (1, 1), (0, 0))).astype(BF16)
    if C >= 32:
        hp = out_hw + 1
        pl4 = xp.reshape(B, hp, 2, hp, 2, C).transpose(0, 2, 4, 1, 3, 5)
        cols = [
            pl4[:, dy % 2, dx % 2,
                dy // 2 : dy // 2 + out_hw, dx // 2 : dx // 2 + out_hw, :]
            for dy in range(4) for dx in range(4)
        ]
    else:
        s = 2 * out_hw - 1
        cols = [xp[:, dy : dy + s : 2, dx : dx + s : 2, :]
                for dy in range(4) for dx in range(4)]
    return jnp.concatenate(cols, axis=-1).reshape(B * out_hw * out_hw, 16 * C)


def _w_s2k4(w):
    O, C = w.shape[0], w.shape[1]
    return w.transpose(2, 3, 1, 0).reshape(16 * C, O).astype(BF16)


def _patches_s1k3(x_nhwc, out_hw):
    B, H, W, C = x_nhwc.shape
    xp = jnp.pad(x_nhwc, ((0, 0), (1, 1), (1, 1), (0, 0))).astype(BF16)
    cols = [xp[:, dy : dy + out_hw, dx : dx + out_hw, :]
            for dy in range(3) for dx in range(3)]
    return jnp.concatenate(cols, axis=-1).reshape(B * out_hw * out_hw, 9 * C)


def _w_s1k3(w):
    O, C = w.shape[0], w.shape[1]
    return w.transpose(2, 3, 1, 0).reshape(9 * C, O).astype(BF16)


_TAPS = ((3, 1), (2, 0))


def _w_convt(w, n_pad):
    cin, cout = w.shape[0], w.shape[1]
    blocks = []
    for dy in range(3):
        for dx in range(3):
            phase_cols = []
            for py in range(2):
                for px in range(2):
                    wy, wx = dy - py, dx - px
                    if 0 <= wy <= 1 and 0 <= wx <= 1:
                        sub = w[:, :, _TAPS[py][wy], _TAPS[px][wx]]
                    else:
                        sub = jnp.zeros((cin, cout), F32)
                    phase_cols.append(sub)
            blocks.append(jnp.concatenate(phase_cols, axis=1))
    wm = jnp.concatenate(blocks, axis=0)
    if n_pad > 4 * cout:
        wm = jnp.pad(wm, ((0, 0), (0, n_pad - 4 * cout)))
    return wm.astype(BF16)


def _interleave_phases(y, B, hin, cout):
    y = y[:, : 4 * cout].reshape(B, hin, hin, 2, 2, cout)
    y = y.transpose(0, 1, 3, 2, 4, 5)
    return y.reshape(B, 2 * hin, 2 * hin, cout)


def kernel(x, w1, b1, w2, b2, w3, b3, w4, b4, codebooks, dw0, db0,
           dtw1, dtb1, dtw2, dtb2):
    B = x.shape[0]
    xh = x.transpose(0, 2, 3, 1)

    y1 = _mm(_patches_s2k4(xh, 112), _w_s2k4(w1), b1, act="relu", mb=1792)
    y1 = y1.reshape(B, 112, 112, 64)

    y2 = _mm(_patches_s2k4(y1, 56), _w_s2k4(w2), b2, act="relu", mb=896)
    y2 = y2.reshape(B, 56, 56, 128)

    y3 = _mm(_patches_s1k3(y2, 56), _w_s1k3(w3), b3, act="relu", mb=896)
    y3 = y3.reshape(B, 56, 56, 256)

    tokens = _mm(_patches_s1k3(y3, 56), _w_s1k3(w4), b4, act="none", mb=896)

    quant, quant_t, idx, loss_part = _vq(tokens, codebooks, tb=896)
    T, C = tokens.shape
    commit_loss = loss_part.sum(axis=1) / (T * C)
    indices = idx.reshape(4, B, 56, 56).transpose(1, 0, 2, 3)

    qmap_nhwc = quant.reshape(B, 56, 56, C)
    qmap = quant_t.reshape(C, B, 56, 56).transpose(1, 0, 2, 3)

    r0 = _mm(_patches_s1k3(qmap_nhwc, 56), _w_s1k3(dw0), db0, act="relu",
             mb=896)
    r0 = r0.reshape(B, 56, 56, 128)

    bt1 = jnp.pad(jnp.tile(dtb1, 4), (0, 0))
    r1 = _mm(_patches_s1k3(r0, 56), _w_convt(dtw1, 256), bt1, act="relu",
             mb=896)
    r1 = _interleave_phases(r1, B, 56, 64)

    bt2 = jnp.pad(jnp.tile(dtb2, 4), (0, 116))
    r2 = _mm(_patches_s1k3(r1, 112), _w_convt(dtw2, 128), bt2, act="tanh",
             mb=1792)
    recon = _interleave_phases(r2, B, 112, 3)
    recon = recon.transpose(0, 3, 1, 2)

    return recon, indices, commit_loss, qmap


# speedup vs baseline: 3.0482x; 1.1061x over previous
import functools

import jax
import jax.numpy as jnp
from jax import lax
from jax.experimental import pallas as pl

F32 = jnp.float32
BF16 = jnp.bfloat16


def _dot(a, b, trans_b=False, exact=False):
    dn = (((1,), (1 if trans_b else 0,)), ((), ()))
    if exact:
        return lax.dot_general(a, b, dn, precision=lax.Precision.HIGHEST,
                               preferred_element_type=F32)
    return lax.dot_general(a.astype(BF16), b.astype(BF16), dn,
                           preferred_element_type=F32)


def _mm_body(a_ref, w_ref, b_ref, o_ref, *, act):
    y = _dot(a_ref[...], w_ref[...]) + b_ref[...]
    if act == "relu":
        y = jnp.maximum(y, 0.0)
    elif act == "tanh":
        y = jnp.tanh(y)
    o_ref[...] = y


def _mm(a, w, bias, act="none", mb=1792):
    M, K = a.shape
    N = w.shape[1]
    nblk = M // mb
    assert nblk * mb == M
    return pl.pallas_call(
        functools.partial(_mm_body, act=act),
        grid=(nblk,),
        in_specs=[
            pl.BlockSpec((mb, K), lambda i: (i, 0)),
            pl.BlockSpec((K, N), lambda i: (0, 0)),
            pl.BlockSpec((1, N), lambda i: (0, 0)),
        ],
        out_specs=pl.BlockSpec((mb, N), lambda i: (i, 0)),
        out_shape=jax.ShapeDtypeStruct((M, N), F32),
    )(a, w, bias.reshape(1, N))


def _vq_body(tok_ref, cb_ref, cbp_ref, q_ref, qt_ref, idx_ref, loss_ref, *,
             nq, v, tb):
    i = pl.program_id(0)

    @pl.when(i == 0)
    def _init():
        loss_ref[...] = jnp.zeros_like(loss_ref)

    r = tok_ref[...]
    quant = jnp.zeros_like(r)
    iota = lax.broadcasted_iota(jnp.int32, (tb, v), 1)
    for q in range(nq):
        cb = cb_ref[q]
        rn = jnp.sum(r * r, axis=1, keepdims=True)
        cn = jnp.sum(cb * cb, axis=1)[None, :]
        d = rn - 2.0 * _dot(r, cb, trans_b=True) + cn
        idx = jnp.argmin(d, axis=1).astype(jnp.int32)
        oh = (iota == idx[:, None]).astype(BF16)
        dnm = (((1,), (0,)), ((), ()))
        qv = (lax.dot_general(oh, cbp_ref[0, q], dnm, preferred_element_type=F32)
              + lax.dot_general(oh, cbp_ref[1, q], dnm, preferred_element_type=F32)
              + lax.dot_general(oh, cbp_ref[2, q], dnm, preferred_element_type=F32))
        r = r - qv
        quant = quant + qv
        idx_ref[pl.ds(q, 1), :] = idx[None, :]
        part = jnp.sum(r * r, axis=0)
        loss_ref[pl.ds(q, 1), :] += part.reshape(-1, 128).sum(axis=0)[None, :]
    q_ref[...] = quant
    qt_ref[...] = quant.T


def _vq(tokens, codebooks, tb=896):
    T, C = tokens.shape
    NQ, V, _ = codebooks.shape
    nblk = T // tb
    assert nblk * tb == T
    hi = codebooks.astype(BF16)
    mid = (codebooks - hi.astype(F32)).astype(BF16)
    lo = (codebooks - hi.astype(F32) - mid.astype(F32)).astype(BF16)
    cb_parts = jnp.stack([hi, mid, lo])
    return pl.pallas_call(
        functools.partial(_vq_body, nq=NQ, v=V, tb=tb),
        grid=(nblk,),
        in_specs=[
            pl.BlockSpec((tb, C), lambda i: (i, 0)),
            pl.BlockSpec((NQ, V, C), lambda i: (0, 0, 0)),
            pl.BlockSpec((3, NQ, V, C), lambda i: (0, 0, 0, 0)),
        ],
        out_specs=[
            pl.BlockSpec((tb, C), lambda i: (i, 0)),
            pl.BlockSpec((C, tb), lambda i: (0, i)),
            pl.BlockSpec((NQ, tb), lambda i: (0, i)),
            pl.BlockSpec((NQ, 128), lambda i: (0, 0)),
        ],
        out_shape=[
            jax.ShapeDtypeStruct((T, C), F32),
            jax.ShapeDtypeStruct((C, T), F32),
            jax.ShapeDtypeStruct((NQ, T), jnp.int32),
            jax.ShapeDtypeStruct((NQ, 128), F32),
        ],
    )(tokens, codebooks, cb_parts)


def _patches_s2k4(x_nhwc, out_hw):
    B, H, W, C = x_nhwc.shape
    xp = jnp.pad(x_nhwc, ((0, 0), (1, 1), (1, 1), (0, 0))).astype(BF16)
    if C >= 32:
        hp = out_hw + 1
        pl4 = xp.reshape(B, hp, 2, hp, 2, C).transpose(0, 2, 4, 1, 3, 5)
        cols = [
            pl4[:, dy % 2, dx % 2,
                dy // 2 : dy // 2 + out_hw, dx // 2 : dx // 2 + out_hw, :]
            for dy in range(4) for dx in range(4)
        ]
    else:
        s = 2 * out_hw - 1
        cols = [xp[:, dy : dy + s : 2, dx : dx + s : 2, :]
                for dy in range(4) for dx in range(4)]
    return jnp.concatenate(cols, axis=-1).reshape(B * out_hw * out_hw, 16 * C)


def _w_s2k4(w):
    O, C = w.shape[0], w.shape[1]
    return w.transpose(2, 3, 1, 0).reshape(16 * C, O).astype(BF16)


def _patches_s1k3(x_nhwc, out_hw):
    B, H, W, C = x_nhwc.shape
    xp = jnp.pad(x_nhwc, ((0, 0), (1, 1), (1, 1), (0, 0))).astype(BF16)
    cols = [xp[:, dy : dy + out_hw, dx : dx + out_hw, :]
            for dy in range(3) for dx in range(3)]
    return jnp.concatenate(cols, axis=-1).reshape(B * out_hw * out_hw, 9 * C)


def _w_s1k3(w):
    O, C = w.shape[0], w.shape[1]
    return w.transpose(2, 3, 1, 0).reshape(9 * C, O).astype(BF16)


_TAPS = ((3, 1), (2, 0))


def _w_convt(w, n_pad):
    cin, cout = w.shape[0], w.shape[1]
    blocks = []
    for dy in range(3):
        for dx in range(3):
            phase_cols = []
            for py in range(2):
                for px in range(2):
                    wy, wx = dy - py, dx - px
                    if 0 <= wy <= 1 and 0 <= wx <= 1:
                        sub = w[:, :, _TAPS[py][wy], _TAPS[px][wx]]
                    else:
                        sub = jnp.zeros((cin, cout), F32)
                    phase_cols.append(sub)
            blocks.append(jnp.concatenate(phase_cols, axis=1))
    wm = jnp.concatenate(blocks, axis=0)
    if n_pad > 4 * cout:
        wm = jnp.pad(wm, ((0, 0), (0, n_pad - 4 * cout)))
    return wm.astype(BF16)


def _interleave_phases(y, B, hin, cout):
    y = y[:, : 4 * cout].reshape(B, hin, hin, 2, 2, cout)
    y = y.transpose(0, 1, 3, 2, 4, 5)
    return y.reshape(B, 2 * hin, 2 * hin, cout)


def kernel(x, w1, b1, w2, b2, w3, b3, w4, b4, codebooks, dw0, db0,
           dtw1, dtb1, dtw2, dtb2):
    B = x.shape[0]
    xh = x.transpose(0, 2, 3, 1)

    y1 = _mm(_patches_s2k4(xh, 112), _w_s2k4(w1), b1, act="relu", mb=1792)
    y1 = y1.reshape(B, 112, 112, 64)

    y2 = _mm(_patches_s2k4(y1, 56), _w_s2k4(w2), b2, act="relu", mb=896)
    y2 = y2.reshape(B, 56, 56, 128)

    y3 = _mm(_patches_s1k3(y2, 56), _w_s1k3(w3), b3, act="relu", mb=896)
    y3 = y3.reshape(B, 56, 56, 256)

    tokens = _mm(_patches_s1k3(y3, 56), _w_s1k3(w4), b4, act="none", mb=896)

    quant, quant_t, idx, loss_part = _vq(tokens, codebooks, tb=896)
    T, C = tokens.shape
    commit_loss = loss_part.sum(axis=1) / (T * C)
    indices = idx.reshape(4, B, 56, 56).transpose(1, 0, 2, 3)

    qmap_nhwc = quant.reshape(B, 56, 56, C)
    qmap = quant_t.reshape(C, B, 56, 56).transpose(1, 0, 2, 3)

    r0 = _mm(_patches_s1k3(qmap_nhwc, 56), _w_s1k3(dw0), db0, act="relu",
             mb=896)
    r0 = r0.reshape(B, 56, 56, 128)

    bt1 = jnp.pad(jnp.tile(dtb1, 4), (0, 0))
    r1 = _mm(_patches_s1k3(r0, 56), _w_convt(dtw1, 256), bt1, act="relu",
             mb=896)
    r1 = _interleave_phases(r1, B, 56, 64)

    bt2 = jnp.pad(jnp.tile(dtb2, 4), (0, 116))
    r2 = _mm(_patches_s1k3(r1, 112), _w_convt(dtw2, 128), bt2, act="tanh",
             mb=1792)
    recon = _interleave_phases(r2, B, 112, 3)
    recon = recon.transpose(0, 3, 1, 2)

    return recon, indices, commit_loss, qmap
